# Initial kernel scaffold; baseline (speedup 1.0000x reference)
import functools

import jax
import jax.numpy as jnp
from jax import lax
from jax.experimental import pallas as pl
from jax.experimental.pallas import tpu as pltpu
from jax.experimental.pallas import tpu_sc as plsc

NC = 2
NS = 16
L = 16
NW = NC * NS

CHUNK = 128


def _sc_segment_sum(n_nodes, d, chunks_per_w):
    rows_per_tile = n_nodes // NS
    edges_per_w = chunks_per_w * CHUNK
    mesh = plsc.VectorSubcoreMesh(core_axis_name="c", subcore_axis_name="s")

    @functools.partial(
        pl.kernel,
        out_type=jax.ShapeDtypeStruct((NC, n_nodes, d), jnp.float32),
        mesh=mesh,
        scratch_types=[
            pltpu.VMEM((CHUNK,), jnp.int32),
            pltpu.VMEM((CHUNK,), jnp.int32),
            pltpu.VMEM((CHUNK,), jnp.float32),
            pltpu.VMEM((CHUNK, d), jnp.float32),
            pltpu.VMEM_SHARED((n_nodes, d), jnp.float32),
            pltpu.SemaphoreType.DMA,
        ],
    )
    def k(x_hbm, src_hbm, dst_hbm, val_hbm, zeros_hbm, out_hbm,
          src_v, dst_v, val_v, rows_v, h_sh, sem):
        c = lax.axis_index("c")
        s = lax.axis_index("s")
        wid = s * NC + c

        row0 = s * rows_per_tile
        pltpu.sync_copy(zeros_hbm.at[pl.ds(row0, rows_per_tile)],
                        h_sh.at[pl.ds(row0, rows_per_tile)])
        plsc.subcore_barrier()

        def chunk_body(j, carry):
            base = wid * edges_per_w + j * CHUNK
            pltpu.sync_copy(src_hbm.at[pl.ds(base, CHUNK)], src_v)
            pltpu.sync_copy(dst_hbm.at[pl.ds(base, CHUNK)], dst_v)
            pltpu.sync_copy(val_hbm.at[pl.ds(base, CHUNK)], val_v)
            pltpu.async_copy(x_hbm.at[src_v], rows_v, sem).wait()

            def row_body(r, carry2):
                splat = plsc.load_gather(
                    val_v, [jnp.full((L,), 0, jnp.int32) + r])
                for g in range(d // L):
                    sl = pl.ds(g * L, L)
                    rows_v[r, sl] = rows_v[r, sl] * splat
                return carry2

            lax.fori_loop(0, CHUNK, row_body, 0, unroll=2)
            pltpu.sync_copy(rows_v, h_sh.at[dst_v], add=True)
            return carry

        lax.fori_loop(0, chunks_per_w, chunk_body, 0)
        plsc.subcore_barrier()
        pltpu.sync_copy(h_sh.at[pl.ds(row0, rows_per_tile)],
                        out_hbm.at[c, pl.ds(row0, rows_per_tile)])

    return k


def _tc_linear(n_nodes, d, bm=1000):
    def body(p_ref, w_ref, b_ref, o_ref):
        h = p_ref[0] + p_ref[1]
        o_ref[...] = jnp.dot(
            h, w_ref[...].T, preferred_element_type=jnp.float32,
            precision=lax.Precision.HIGHEST) + b_ref[...]

    return pl.pallas_call(
        body,
        grid=(n_nodes // bm,),
        in_specs=[
            pl.BlockSpec((NC, bm, d), lambda i: (0, i, 0)),
            pl.BlockSpec((d, d), lambda i: (0, 0)),
            pl.BlockSpec((1, d), lambda i: (0, 0)),
        ],
        out_specs=pl.BlockSpec((bm, d), lambda i: (i, 0)),
        out_shape=jax.ShapeDtypeStruct((n_nodes, d), jnp.float32),
    )


def kernel(X, edge_index, edge_values, W, b):
    n_nodes, d = X.shape
    n_edges = edge_index.shape[1]
    chunks_per_w = -(-n_edges // (NW * CHUNK))
    e_pad = NW * chunks_per_w * CHUNK

    dst = edge_index[0].astype(jnp.int32)
    src = edge_index[1].astype(jnp.int32)
    pad = e_pad - n_edges
    if pad:
        src = jnp.concatenate([src, jnp.zeros((pad,), jnp.int32)])
        dst = jnp.concatenate([dst, jnp.zeros((pad,), jnp.int32)])
        edge_values = jnp.concatenate(
            [edge_values, jnp.zeros((pad,), jnp.float32)])
    zeros = jnp.zeros((n_nodes, d), jnp.float32)

    partials = _sc_segment_sum(n_nodes, d, chunks_per_w)(
        X, src, dst, edge_values, zeros)
    return _tc_linear(n_nodes, d)(partials, W, jnp.reshape(b, (1, d)))


# speedup vs baseline: 4.1176x; 4.1176x over previous
import functools

import jax
import jax.numpy as jnp
from jax import lax
from jax.experimental import pallas as pl
from jax.experimental.pallas import tpu as pltpu
from jax.experimental.pallas import tpu_sc as plsc

NC = 2
NS = 16
L = 16
NW = NC * NS

CHUNK = 128


def _sc_segment_sum(n_nodes, d, chunks_per_w):
    rows_per_tile = n_nodes // NS
    edges_per_w = chunks_per_w * CHUNK
    mesh = plsc.VectorSubcoreMesh(core_axis_name="c", subcore_axis_name="s")

    @functools.partial(
        pl.kernel,
        out_type=jax.ShapeDtypeStruct((NC, n_nodes, d), jnp.float32),
        mesh=mesh,
        scratch_types=[
            pltpu.VMEM((CHUNK,), jnp.int32),
            pltpu.VMEM((CHUNK,), jnp.int32),
            pltpu.VMEM((CHUNK,), jnp.float32),
            pltpu.VMEM((CHUNK, d), jnp.float32),
            pltpu.VMEM_SHARED((n_nodes, d), jnp.float32),
            pltpu.SemaphoreType.DMA,
        ],
    )
    def k(x_hbm, src_hbm, dst_hbm, val_hbm, zeros_hbm, out_hbm,
          src_v, dst_v, val_v, rows_v, h_sh, sem):
        c = lax.axis_index("c")
        s = lax.axis_index("s")
        wid = s * NC + c

        row0 = s * rows_per_tile
        pltpu.sync_copy(zeros_hbm.at[pl.ds(row0, rows_per_tile)],
                        h_sh.at[pl.ds(row0, rows_per_tile)])
        plsc.subcore_barrier()

        def chunk_body(j, carry):
            base = wid * edges_per_w + j * CHUNK
            pltpu.sync_copy(src_hbm.at[pl.ds(base, CHUNK)], src_v)
            pltpu.sync_copy(dst_hbm.at[pl.ds(base, CHUNK)], dst_v)
            pltpu.sync_copy(val_hbm.at[pl.ds(base, CHUNK)], val_v)
            pltpu.async_copy(x_hbm.at[src_v], rows_v, sem).wait()

            def grp_body(q, carry2):
                vals16 = val_v[pl.ds(q * L, L)]
                for i in range(L):
                    r = q * L + i
                    splat = jnp.full((L,), 0.0, jnp.float32) + vals16[i]
                    for g in range(d // L):
                        sl = pl.ds(g * L, L)
                        rows_v[r, sl] = rows_v[r, sl] * splat
                return carry2

            lax.fori_loop(0, CHUNK // L, grp_body, 0)
            pltpu.sync_copy(rows_v, h_sh.at[dst_v], add=True)
            return carry

        lax.fori_loop(0, chunks_per_w, chunk_body, 0)
        plsc.subcore_barrier()
        pltpu.sync_copy(h_sh.at[pl.ds(row0, rows_per_tile)],
                        out_hbm.at[c, pl.ds(row0, rows_per_tile)])

    return k


def _tc_linear(n_nodes, d, bm=1000):
    def body(p_ref, w_ref, b_ref, o_ref):
        h = p_ref[0] + p_ref[1]
        o_ref[...] = jnp.dot(
            h, w_ref[...].T, preferred_element_type=jnp.float32,
            precision=lax.Precision.HIGHEST) + b_ref[...]

    return pl.pallas_call(
        body,
        grid=(n_nodes // bm,),
        in_specs=[
            pl.BlockSpec((NC, bm, d), lambda i: (0, i, 0)),
            pl.BlockSpec((d, d), lambda i: (0, 0)),
            pl.BlockSpec((1, d), lambda i: (0, 0)),
        ],
        out_specs=pl.BlockSpec((bm, d), lambda i: (i, 0)),
        out_shape=jax.ShapeDtypeStruct((n_nodes, d), jnp.float32),
    )


def kernel(X, edge_index, edge_values, W, b):
    n_nodes, d = X.shape
    n_edges = edge_index.shape[1]
    chunks_per_w = -(-n_edges // (NW * CHUNK))
    e_pad = NW * chunks_per_w * CHUNK
    n_pad = (-(-n_nodes // (8 * NS))) * 8 * NS

    dst = edge_index[0].astype(jnp.int32)
    src = edge_index[1].astype(jnp.int32)
    pad = e_pad - n_edges
    if pad:
        src = jnp.concatenate([src, jnp.zeros((pad,), jnp.int32)])
        dst = jnp.concatenate([dst, jnp.zeros((pad,), jnp.int32)])
        edge_values = jnp.concatenate(
            [edge_values, jnp.zeros((pad,), jnp.float32)])
    zeros = jnp.zeros((n_pad, d), jnp.float32)

    partials = _sc_segment_sum(n_pad, d, chunks_per_w)(
        X, src, dst, edge_values, zeros)
    bm = n_pad // 8
    out = _tc_linear(n_pad, d, bm=bm)(partials, W, jnp.reshape(b, (1, d)))
    return out[:n_nodes]


# speedup vs baseline: 7.8955x; 1.9175x over previous
import functools

import jax
import jax.numpy as jnp
from jax import lax
from jax.experimental import pallas as pl
from jax.experimental.pallas import tpu as pltpu
from jax.experimental.pallas import tpu_sc as plsc

NC = 2
NS = 16
L = 16
NW = NC * NS

CHUNK = 112
NBUF = 3
NIDX = 6


def _sc_segment_sum(n_nodes, d, chunks_per_w):
    rows_per_tile = n_nodes // NS
    edges_per_w = chunks_per_w * CHUNK
    mesh = plsc.VectorSubcoreMesh(core_axis_name="c", subcore_axis_name="s")

    @functools.partial(
        pl.kernel,
        out_type=jax.ShapeDtypeStruct((NC, n_nodes, d), jnp.float32),
        mesh=mesh,
        scratch_types=[
            pltpu.VMEM((NIDX, CHUNK), jnp.int32),
            pltpu.VMEM((NIDX, CHUNK), jnp.int32),
            pltpu.VMEM((NIDX, CHUNK), jnp.float32),
            pltpu.VMEM((NBUF, CHUNK, d), jnp.float32),
            pltpu.VMEM_SHARED((n_nodes, d), jnp.float32),
            [pltpu.SemaphoreType.DMA] * NBUF,
            [pltpu.SemaphoreType.DMA] * NBUF,
            [pltpu.SemaphoreType.DMA] * NIDX,
        ],
    )
    def k(x_hbm, src_hbm, dst_hbm, val_hbm, zeros_hbm, out_hbm,
          src_r, dst_r, val_r, bufs, h_sh, sg, ss, si):
        c = lax.axis_index("c")
        s = lax.axis_index("s")
        wid = s * NC + c
        row0 = s * rows_per_tile
        ebase = wid * edges_per_w

        def idx_load(chunk_i, slot):
            off = ebase + chunk_i * CHUNK
            pltpu.async_copy(src_hbm.at[pl.ds(off, CHUNK)],
                             src_r.at[slot], si[slot])
            pltpu.async_copy(dst_hbm.at[pl.ds(off, CHUNK)],
                             dst_r.at[slot], si[slot])
            pltpu.async_copy(val_hbm.at[pl.ds(off, CHUNK)],
                             val_r.at[slot], si[slot])

        def idx_wait(slot):
            pltpu.make_async_copy(src_hbm.at[pl.ds(0, CHUNK)],
                                  src_r.at[slot], si[slot]).wait()
            pltpu.make_async_copy(dst_hbm.at[pl.ds(0, CHUNK)],
                                  dst_r.at[slot], si[slot]).wait()
            pltpu.make_async_copy(val_hbm.at[pl.ds(0, CHUNK)],
                                  val_r.at[slot], si[slot]).wait()

        def gather_issue(slot, b):
            pltpu.async_copy(x_hbm.at[src_r.at[slot]], bufs.at[b], sg[b])

        def gather_wait(slot, b):
            pltpu.make_async_copy(x_hbm.at[src_r.at[slot]],
                                  bufs.at[b], sg[b]).wait()

        def scatter_issue(slot, b):
            pltpu.async_copy(bufs.at[b], h_sh.at[dst_r.at[slot]],
                             ss[b], add=True)

        def scatter_wait(slot, b):
            pltpu.make_async_copy(bufs.at[b], h_sh.at[dst_r.at[slot]],
                                  ss[b]).wait()

        pltpu.sync_copy(zeros_hbm.at[pl.ds(row0, rows_per_tile)],
                        h_sh.at[pl.ds(row0, rows_per_tile)])
        plsc.subcore_barrier()

        for ci in range(4):
            idx_load(ci, ci)
        idx_wait(0)
        gather_issue(0, 0)

        def pipe_body(j6, carry):
            for u in range(NIDX):
                jj = j6 * NIDX + u
                b = u % NBUF

                @pl.when(jj >= 2)
                def _():
                    scatter_wait((u + 4) % NIDX, (u + 1) % NBUF)

                @pl.when(jj + 4 < chunks_per_w)
                def _():
                    idx_load(jj + 4, (u + 4) % NIDX)

                @pl.when(jj + 1 < chunks_per_w)
                def _():
                    idx_wait((u + 1) % NIDX)
                    gather_issue((u + 1) % NIDX, (u + 1) % NBUF)

                gather_wait(u, b)

                def grp_body(q, carry2):
                    vals16 = val_r[u, pl.ds(q * L, L)]
                    for i in range(L):
                        r = q * L + i
                        splat = jnp.full((L,), 0.0, jnp.float32) + vals16[i]
                        for g in range(d // L):
                            sl = pl.ds(g * L, L)
                            bufs[b, r, sl] = bufs[b, r, sl] * splat
                    return carry2

                lax.fori_loop(0, CHUNK // L, grp_body, 0)
                scatter_issue(u, b)
            return carry

        lax.fori_loop(0, chunks_per_w // NIDX, pipe_body, 0)
        scatter_wait((chunks_per_w - 2) % NIDX, (chunks_per_w - 2) % NBUF)
        scatter_wait((chunks_per_w - 1) % NIDX, (chunks_per_w - 1) % NBUF)
        plsc.subcore_barrier()
        pltpu.sync_copy(h_sh.at[pl.ds(row0, rows_per_tile)],
                        out_hbm.at[c, pl.ds(row0, rows_per_tile)])

    return k


def _tc_linear(n_nodes, d, bm):
    def body(p_ref, w_ref, b_ref, o_ref):
        h = p_ref[0] + p_ref[1]
        o_ref[...] = jnp.dot(
            h, w_ref[...].T, preferred_element_type=jnp.float32,
            precision=lax.Precision.HIGHEST) + b_ref[...]

    return pl.pallas_call(
        body,
        grid=(n_nodes // bm,),
        in_specs=[
            pl.BlockSpec((NC, bm, d), lambda i: (0, i, 0)),
            pl.BlockSpec((d, d), lambda i: (0, 0)),
            pl.BlockSpec((1, d), lambda i: (0, 0)),
        ],
        out_specs=pl.BlockSpec((bm, d), lambda i: (i, 0)),
        out_shape=jax.ShapeDtypeStruct((n_nodes, d), jnp.float32),
    )


def kernel(X, edge_index, edge_values, W, b):
    n_nodes, d = X.shape
    n_edges = edge_index.shape[1]
    chunks_per_w = -(-n_edges // (NW * CHUNK))
    chunks_per_w = -(-chunks_per_w // NIDX) * NIDX
    e_pad = NW * chunks_per_w * CHUNK
    n_pad = (-(-n_nodes // (8 * NS))) * 8 * NS

    dst = edge_index[0].astype(jnp.int32)
    src = edge_index[1].astype(jnp.int32)
    pad = e_pad - n_edges
    if pad:
        src = jnp.concatenate([src, jnp.zeros((pad,), jnp.int32)])
        dst = jnp.concatenate([dst, jnp.zeros((pad,), jnp.int32)])
        edge_values = jnp.concatenate(
            [edge_values, jnp.zeros((pad,), jnp.float32)])
    zeros = jnp.zeros((n_pad, d), jnp.float32)

    partials = _sc_segment_sum(n_pad, d, chunks_per_w)(
        X, src, dst, edge_values, zeros)
    out = _tc_linear(n_pad, d, bm=n_pad // 8)(
        partials, W, jnp.reshape(b, (1, d)))
    return out[:n_nodes]


# speedup vs baseline: 12.0052x; 1.5205x over previous
import functools

import jax
import jax.numpy as jnp
from jax import lax
from jax.experimental import pallas as pl
from jax.experimental.pallas import tpu as pltpu
from jax.experimental.pallas import tpu_sc as plsc

NC = 2
NS = 16
L = 16
NW = NC * NS

CHUNK = 112
NBUF = 3
NIDX = 6


def _sc_segment_sum(n_nodes, d, chunks_per_w):
    rows_per_tile = n_nodes // NS
    edges_per_w = chunks_per_w * CHUNK
    mesh = plsc.VectorSubcoreMesh(core_axis_name="c", subcore_axis_name="s")

    @functools.partial(
        pl.kernel,
        out_type=jax.ShapeDtypeStruct((NC, n_nodes, d), jnp.float32),
        mesh=mesh,
        scratch_types=[
            pltpu.VMEM((NIDX, CHUNK), jnp.int32),
            pltpu.VMEM((NIDX, CHUNK), jnp.int32),
            pltpu.VMEM((NIDX, CHUNK), jnp.float32),
            pltpu.VMEM((NBUF, CHUNK, d), jnp.float32),
            pltpu.VMEM_SHARED((n_nodes, d), jnp.float32),
            [pltpu.SemaphoreType.DMA] * NBUF,
            [pltpu.SemaphoreType.DMA] * NBUF,
            [pltpu.SemaphoreType.DMA] * NIDX,
        ],
    )
    def k(x_hbm, src_hbm, dst_hbm, val_hbm, zeros_hbm, out_hbm,
          src_r, dst_r, val_r, bufs, h_sh, sg, ss, si):
        c = lax.axis_index("c")
        s = lax.axis_index("s")
        wid = s * NC + c
        row0 = s * rows_per_tile
        ebase = wid * edges_per_w

        def idx_load(chunk_i, slot):
            off = ebase + chunk_i * CHUNK
            pltpu.async_copy(src_hbm.at[pl.ds(off, CHUNK)],
                             src_r.at[slot], si[slot])
            pltpu.async_copy(dst_hbm.at[pl.ds(off, CHUNK)],
                             dst_r.at[slot], si[slot])
            pltpu.async_copy(val_hbm.at[pl.ds(off, CHUNK)],
                             val_r.at[slot], si[slot])

        def idx_wait(slot):
            pltpu.make_async_copy(src_hbm.at[pl.ds(0, CHUNK)],
                                  src_r.at[slot], si[slot]).wait()
            pltpu.make_async_copy(dst_hbm.at[pl.ds(0, CHUNK)],
                                  dst_r.at[slot], si[slot]).wait()
            pltpu.make_async_copy(val_hbm.at[pl.ds(0, CHUNK)],
                                  val_r.at[slot], si[slot]).wait()

        def gather_issue(slot, b):
            pltpu.async_copy(x_hbm.at[src_r.at[slot]], bufs.at[b], sg[b])

        def gather_wait(slot, b):
            pltpu.make_async_copy(x_hbm.at[src_r.at[slot]],
                                  bufs.at[b], sg[b]).wait()

        def scatter_issue(slot, b):
            pltpu.async_copy(bufs.at[b], h_sh.at[dst_r.at[slot]],
                             ss[b], add=True)

        def scatter_wait(slot, b):
            pltpu.make_async_copy(bufs.at[b], h_sh.at[dst_r.at[slot]],
                                  ss[b]).wait()

        pltpu.sync_copy(zeros_hbm.at[pl.ds(row0, rows_per_tile)],
                        h_sh.at[pl.ds(row0, rows_per_tile)])
        plsc.subcore_barrier()

        for ci in range(4):
            idx_load(ci, ci)
        idx_wait(0)
        gather_issue(0, 0)

        def pipe_body(j6, carry):
            for u in range(NIDX):
                jj = j6 * NIDX + u
                b = u % NBUF

                @pl.when(jj >= 2)
                def _():
                    scatter_wait((u + 4) % NIDX, (u + 1) % NBUF)

                @pl.when(jj + 4 < chunks_per_w)
                def _():
                    idx_load(jj + 4, (u + 4) % NIDX)

                @pl.when(jj + 1 < chunks_per_w)
                def _():
                    idx_wait((u + 1) % NIDX)
                    gather_issue((u + 1) % NIDX, (u + 1) % NBUF)

                gather_wait(u, b)

                def grp_body(q, carry2):
                    vals16 = val_r[u, pl.ds(q * L, L)]
                    for i in range(L):
                        r = q * L + i
                        splat = jnp.full((L,), 0.0, jnp.float32) + vals16[i]
                        for g in range(d // L):
                            sl = pl.ds(g * L, L)
                            bufs[b, r, sl] = bufs[b, r, sl] * splat
                    return carry2

                lax.fori_loop(0, CHUNK // L, grp_body, 0)
                scatter_issue(u, b)
            return carry

        lax.fori_loop(0, chunks_per_w // NIDX, pipe_body, 0)
        scatter_wait((chunks_per_w - 2) % NIDX, (chunks_per_w - 2) % NBUF)
        scatter_wait((chunks_per_w - 1) % NIDX, (chunks_per_w - 1) % NBUF)
        plsc.subcore_barrier()
        pltpu.sync_copy(h_sh.at[pl.ds(row0, rows_per_tile)],
                        out_hbm.at[c, pl.ds(row0, rows_per_tile)])

    return k


def _tc_linear(n_nodes, d, bm):
    def body(p_ref, w_ref, b_ref, o_ref):
        h = p_ref[0] + p_ref[1]
        o_ref[...] = jnp.dot(
            h, w_ref[...].T, preferred_element_type=jnp.float32,
            precision=lax.Precision.HIGHEST) + b_ref[...]

    return pl.pallas_call(
        body,
        grid=(n_nodes // bm,),
        in_specs=[
            pl.BlockSpec((NC, bm, d), lambda i: (0, i, 0)),
            pl.BlockSpec((d, d), lambda i: (0, 0)),
            pl.BlockSpec((1, d), lambda i: (0, 0)),
        ],
        out_specs=pl.BlockSpec((bm, d), lambda i: (i, 0)),
        out_shape=jax.ShapeDtypeStruct((n_nodes, d), jnp.float32),
    )


def kernel(X, edge_index, edge_values, W, b):
    n_nodes, d = X.shape
    n_edges = edge_index.shape[1]
    chunks_per_w = -(-n_edges // (NW * CHUNK))
    chunks_per_w = -(-chunks_per_w // NIDX) * NIDX
    e_pad = NW * chunks_per_w * CHUNK
    n_pad = (-(-n_nodes // (8 * NS))) * 8 * NS

    dst = edge_index[0].astype(jnp.int32)
    src = edge_index[1].astype(jnp.int32)
    pad = e_pad - n_edges
    if pad:
        spread = jnp.arange(pad, dtype=jnp.int32) % n_nodes
        src = jnp.concatenate([src, spread])
        dst = jnp.concatenate([dst, spread])
        edge_values = jnp.concatenate(
            [edge_values, jnp.zeros((pad,), jnp.float32)])
    zeros = jnp.zeros((n_pad, d), jnp.float32)

    partials = _sc_segment_sum(n_pad, d, chunks_per_w)(
        X, src, dst, edge_values, zeros)
    out = _tc_linear(n_pad, d, bm=n_pad // 8)(
        partials, W, jnp.reshape(b, (1, d)))
    return out[:n_nodes]


# speedup vs baseline: 12.6208x; 1.0513x over previous
import functools

import jax
import jax.numpy as jnp
from jax import lax
from jax.experimental import pallas as pl
from jax.experimental.pallas import tpu as pltpu
from jax.experimental.pallas import tpu_sc as plsc

NC = 2
NS = 16
L = 16
NW = NC * NS

CHUNK = 80
NBUF = 4
NIDX = 8


def _sc_segment_sum(n_nodes, d, n_edges, chunks_per_w):
    rows_per_tile = n_nodes // NS
    edges_per_w = chunks_per_w * CHUNK
    main_iters = (chunks_per_w - 5) // NIDX
    tail_start = main_iters * NIDX
    mesh = plsc.VectorSubcoreMesh(core_axis_name="c", subcore_axis_name="s")

    @functools.partial(
        pl.kernel,
        out_type=jax.ShapeDtypeStruct((NC, n_nodes, d), jnp.float32),
        mesh=mesh,
        scratch_types=[
            pltpu.VMEM((NIDX, CHUNK), jnp.int32),
            pltpu.VMEM((NIDX, CHUNK), jnp.int32),
            pltpu.VMEM((NIDX, CHUNK), jnp.float32),
            pltpu.VMEM((NBUF, CHUNK, d), jnp.float32),
            pltpu.VMEM_SHARED((n_nodes, d), jnp.float32),
            [pltpu.SemaphoreType.DMA] * NBUF,
            [pltpu.SemaphoreType.DMA] * NBUF,
            [pltpu.SemaphoreType.DMA] * NIDX,
        ],
    )
    def k(x_hbm, eidx_hbm, val_hbm, zeros_hbm, out_hbm,
          src_r, dst_r, val_r, bufs, h_sh, sg, ss, si):
        c = lax.axis_index("c")
        s = lax.axis_index("s")
        wid = s * NC + c
        row0 = s * rows_per_tile
        ebase = wid * edges_per_w

        def idx_load(chunk_i, slot):
            off = ebase + chunk_i * CHUNK
            pltpu.async_copy(eidx_hbm.at[pl.ds(n_edges + off, CHUNK)],
                             src_r.at[slot], si[slot])
            pltpu.async_copy(eidx_hbm.at[pl.ds(off, CHUNK)],
                             dst_r.at[slot], si[slot])
            pltpu.async_copy(val_hbm.at[pl.ds(off, CHUNK)],
                             val_r.at[slot], si[slot])

        def idx_wait(slot):
            pltpu.make_async_copy(eidx_hbm.at[pl.ds(0, CHUNK)],
                                  src_r.at[slot], si[slot]).wait()
            pltpu.make_async_copy(eidx_hbm.at[pl.ds(0, CHUNK)],
                                  dst_r.at[slot], si[slot]).wait()
            pltpu.make_async_copy(val_hbm.at[pl.ds(0, CHUNK)],
                                  val_r.at[slot], si[slot]).wait()

        def gather_issue(slot, b):
            pltpu.async_copy(x_hbm.at[src_r.at[slot]], bufs.at[b], sg[b])

        def gather_wait(slot, b):
            pltpu.make_async_copy(x_hbm.at[src_r.at[slot]],
                                  bufs.at[b], sg[b]).wait()

        def scatter_issue(slot, b):
            pltpu.async_copy(bufs.at[b], h_sh.at[dst_r.at[slot]],
                             ss[b], add=True)

        def scatter_wait(slot, b):
            pltpu.make_async_copy(bufs.at[b], h_sh.at[dst_r.at[slot]],
                                  ss[b]).wait()

        def compute(u, b):
            def grp_body(q, carry2):
                vals16 = val_r[u, pl.ds(q * L, L)]
                for i in range(L):
                    r = q * L + i
                    splat = jnp.full((L,), 0.0, jnp.float32) + vals16[i]
                    for g in range(d // L):
                        sl = pl.ds(g * L, L)
                        bufs[b, r, sl] = bufs[b, r, sl] * splat
                return carry2

            lax.fori_loop(0, CHUNK // L, grp_body, 0)

        def emit_iter(jj, u, tail):
            b = u % NBUF
            if tail:
                scatter_wait((u - 3) % NIDX, (b - 3) % NBUF)
            else:
                @pl.when(jj >= 3)
                def _():
                    scatter_wait((u - 3) % NIDX, (b - 3) % NBUF)
            if not tail:
                idx_load(jj + 5, (u + 5) % NIDX)
            if not (tail and u == (chunks_per_w - 1) % NIDX):
                idx_wait((u + 1) % NIDX)
                gather_issue((u + 1) % NIDX, (b + 1) % NBUF)
            gather_wait(u, b)
            compute(u, b)
            scatter_issue(u, b)

        pltpu.sync_copy(zeros_hbm.at[pl.ds(row0, rows_per_tile)],
                        h_sh.at[pl.ds(row0, rows_per_tile)])
        plsc.subcore_barrier()

        for ci in range(5):
            idx_load(ci, ci)
        idx_wait(0)
        gather_issue(0, 0)

        def pipe_body(j8, carry):
            for u in range(NIDX):
                emit_iter(j8 * NIDX + u, u, False)
            return carry

        lax.fori_loop(0, main_iters, pipe_body, 0)
        for jj in range(tail_start, chunks_per_w):
            emit_iter(jj, jj % NIDX, True)
        for jj in range(chunks_per_w - 3, chunks_per_w):
            scatter_wait(jj % NIDX, jj % NBUF)
        plsc.subcore_barrier()
        pltpu.sync_copy(h_sh.at[pl.ds(row0, rows_per_tile)],
                        out_hbm.at[c, pl.ds(row0, rows_per_tile)])

    return k


def _tc_linear(n_out, d, bm):
    def body(p_ref, w_ref, b_ref, o_ref):
        h = p_ref[0] + p_ref[1]
        o_ref[...] = jnp.dot(
            h, w_ref[...].T, preferred_element_type=jnp.float32,
            precision=lax.Precision.HIGHEST) + b_ref[...]

    return pl.pallas_call(
        body,
        grid=(n_out // bm,),
        in_specs=[
            pl.BlockSpec((NC, bm, d), lambda i: (0, i, 0)),
            pl.BlockSpec((d, d), lambda i: (0, 0)),
            pl.BlockSpec((1, d), lambda i: (0, 0)),
        ],
        out_specs=pl.BlockSpec((bm, d), lambda i: (i, 0)),
        out_shape=jax.ShapeDtypeStruct((n_out, d), jnp.float32),
    )


def kernel(X, edge_index, edge_values, W, b):
    n_nodes, d = X.shape
    n_edges = edge_index.shape[1]
    n_pad = (-(-n_nodes // (8 * NS))) * 8 * NS

    eflat = edge_index.astype(jnp.int32).reshape(-1)
    assert n_edges % (NW * CHUNK) == 0, "edge count must tile evenly"
    chunks_per_w = n_edges // (NW * CHUNK)
    zeros = jnp.zeros((n_pad, d), jnp.float32)

    partials = _sc_segment_sum(n_pad, d, n_edges, chunks_per_w)(
        X, eflat, edge_values, zeros)
    return _tc_linear(n_nodes, d, bm=n_nodes // 10)(
        partials, W, jnp.reshape(b, (1, d)))


# speedup vs baseline: 13.2534x; 1.0501x over previous
import functools

import jax
import jax.numpy as jnp
from jax import lax
from jax.experimental import pallas as pl
from jax.experimental.pallas import tpu as pltpu
from jax.experimental.pallas import tpu_sc as plsc

NC = 2
NS = 16
L = 16
NW = NC * NS

CHUNK = 80
NBUF = 4
NIDX = 8


def _sc_segment_sum(n_nodes, d, n_edges, chunks_per_w):
    rows_per_tile = n_nodes // NS
    edges_per_w = chunks_per_w * CHUNK
    main_iters = (chunks_per_w - 5) // NIDX
    tail_start = main_iters * NIDX
    mesh = plsc.VectorSubcoreMesh(core_axis_name="c", subcore_axis_name="s")

    @functools.partial(
        pl.kernel,
        out_type=jax.ShapeDtypeStruct((NC, n_nodes, d), jnp.float32),
        mesh=mesh,
        scratch_types=[
            pltpu.VMEM((NIDX, CHUNK), jnp.int32),
            pltpu.VMEM((NIDX, CHUNK), jnp.int32),
            pltpu.VMEM((NIDX, CHUNK), jnp.float32),
            pltpu.VMEM((NBUF, CHUNK, d), jnp.float32),
            pltpu.VMEM_SHARED((n_nodes, d), jnp.float32),
            [pltpu.SemaphoreType.DMA] * NBUF,
            [pltpu.SemaphoreType.DMA] * NBUF,
            [pltpu.SemaphoreType.DMA] * NIDX,
        ],
    )
    def k(x_hbm, eidx_hbm, val_hbm, zeros_hbm, out_hbm,
          src_r, dst_r, val_r, bufs, h_sh, sg, ss, si):
        c = lax.axis_index("c")
        s = lax.axis_index("s")
        wid = s * NC + c
        row0 = s * rows_per_tile
        ebase = wid * edges_per_w

        def idx_load(chunk_i, slot):
            off = ebase + chunk_i * CHUNK
            pltpu.async_copy(eidx_hbm.at[pl.ds(n_edges + off, CHUNK)],
                             src_r.at[slot], si[slot])
            pltpu.async_copy(eidx_hbm.at[pl.ds(off, CHUNK)],
                             dst_r.at[slot], si[slot])
            pltpu.async_copy(val_hbm.at[pl.ds(off, CHUNK)],
                             val_r.at[slot], si[slot])

        def idx_wait(slot):
            pltpu.make_async_copy(eidx_hbm.at[pl.ds(0, CHUNK)],
                                  src_r.at[slot], si[slot]).wait()
            pltpu.make_async_copy(eidx_hbm.at[pl.ds(0, CHUNK)],
                                  dst_r.at[slot], si[slot]).wait()
            pltpu.make_async_copy(val_hbm.at[pl.ds(0, CHUNK)],
                                  val_r.at[slot], si[slot]).wait()

        def gather_issue(slot, b):
            pltpu.async_copy(x_hbm.at[src_r.at[slot]], bufs.at[b], sg[b])

        def gather_wait(slot, b):
            pltpu.make_async_copy(x_hbm.at[src_r.at[slot]],
                                  bufs.at[b], sg[b]).wait()

        def scatter_issue(slot, b):
            pltpu.async_copy(bufs.at[b], h_sh.at[dst_r.at[slot]],
                             ss[b], add=True)

        def scatter_wait(slot, b):
            pltpu.make_async_copy(bufs.at[b], h_sh.at[dst_r.at[slot]],
                                  ss[b]).wait()

        def compute(u, b):
            def grp_body(q, carry2):
                vals16 = val_r[u, pl.ds(q * L, L)]
                for i in range(L):
                    r = q * L + i
                    splat = lax.gather(
                        vals16, jnp.full((L, 1), i, jnp.int32),
                        lax.GatherDimensionNumbers(
                            offset_dims=(), collapsed_slice_dims=(0,),
                            start_index_map=(0,)),
                        (1,), mode=lax.GatherScatterMode.PROMISE_IN_BOUNDS)
                    for g in range(d // L):
                        sl = pl.ds(g * L, L)
                        bufs[b, r, sl] = bufs[b, r, sl] * splat
                return carry2

            lax.fori_loop(0, CHUNK // L, grp_body, 0)

        def emit_iter(jj, u, tail):
            b = u % NBUF
            if tail:
                scatter_wait((u - 3) % NIDX, (b - 3) % NBUF)
            else:
                @pl.when(jj >= 3)
                def _():
                    scatter_wait((u - 3) % NIDX, (b - 3) % NBUF)
            if not tail:
                idx_load(jj + 5, (u + 5) % NIDX)
            if not (tail and u == (chunks_per_w - 1) % NIDX):
                idx_wait((u + 1) % NIDX)
                gather_issue((u + 1) % NIDX, (b + 1) % NBUF)
            gather_wait(u, b)
            compute(u, b)
            scatter_issue(u, b)

        pltpu.sync_copy(zeros_hbm, h_sh.at[pl.ds(row0, rows_per_tile)])
        plsc.subcore_barrier()

        for ci in range(5):
            idx_load(ci, ci)
        idx_wait(0)
        gather_issue(0, 0)

        def pipe_body(j8, carry):
            for u in range(NIDX):
                emit_iter(j8 * NIDX + u, u, False)
            return carry

        lax.fori_loop(0, main_iters, pipe_body, 0)
        for jj in range(tail_start, chunks_per_w):
            emit_iter(jj, jj % NIDX, True)
        for jj in range(chunks_per_w - 3, chunks_per_w):
            scatter_wait(jj % NIDX, jj % NBUF)
        plsc.subcore_barrier()
        pltpu.sync_copy(h_sh.at[pl.ds(row0, rows_per_tile)],
                        out_hbm.at[c, pl.ds(row0, rows_per_tile)])

    return k


def _tc_linear(n_out, d, bm):
    def body(p_ref, w_ref, b_ref, o_ref):
        h = p_ref[0] + p_ref[1]
        o_ref[...] = jnp.dot(
            h, w_ref[...].T, preferred_element_type=jnp.float32) + b_ref[...]

    return pl.pallas_call(
        body,
        grid=(n_out // bm,),
        in_specs=[
            pl.BlockSpec((NC, bm, d), lambda i: (0, i, 0)),
            pl.BlockSpec((d, d), lambda i: (0, 0)),
            pl.BlockSpec((1, d), lambda i: (0, 0)),
        ],
        out_specs=pl.BlockSpec((bm, d), lambda i: (i, 0)),
        out_shape=jax.ShapeDtypeStruct((n_out, d), jnp.float32),
    )


def kernel(X, edge_index, edge_values, W, b):
    n_nodes, d = X.shape
    n_edges = edge_index.shape[1]
    n_pad = (-(-n_nodes // (8 * NS))) * 8 * NS

    eflat = edge_index.astype(jnp.int32).reshape(-1)
    assert n_edges % (NW * CHUNK) == 0, "edge count must tile evenly"
    chunks_per_w = n_edges // (NW * CHUNK)
    zeros = jnp.zeros((n_pad // NS, d), jnp.float32)

    partials = _sc_segment_sum(n_pad, d, n_edges, chunks_per_w)(
        X, eflat, edge_values, zeros)
    return _tc_linear(n_nodes, d, bm=n_nodes // 5)(
        partials, W, jnp.reshape(b, (1, d)))
